# baseline (speedup 1.0000x reference)
import functools

import jax
import jax.numpy as jnp
from jax import lax
from jax.experimental import pallas as pl
from jax.experimental.pallas import tpu as pltpu
from jax.experimental.pallas import tpu_sc as plsc

NC = 2
NS = 16
LN = 16
CH = 80


def _tc_transform(x, bases, coeff, W_self):
    N, D = x.shape
    Bb = bases.shape[0]
    Rr = coeff.shape[0]
    Do = W_self.shape[1]
    BN = 1000
    Wtot = Do * (Rr + 1)

    def body(x_ref, b_ref, c_ref, w_ref, h_ref, z_ref, wcat_ref):
        @pl.when(pl.program_id(0) == 0)
        def _():
            wcat_ref[:, 0:Do] = w_ref[...]
            c = c_ref[...]
            for r in range(Rr):
                acc = c[r, 0] * b_ref[0]
                for b in range(1, Bb):
                    acc = acc + c[r, b] * b_ref[b]
                wcat_ref[:, (r + 1) * Do:(r + 2) * Do] = acc

        y = jnp.dot(x_ref[...], wcat_ref[...], preferred_element_type=jnp.float32)
        h_ref[...] = y[:, 0:Do]
        zb = y[:, Do:].astype(jnp.bfloat16)
        u32 = lax.bitcast_convert_type(zb, jnp.uint16).astype(jnp.uint32)
        hw = Do // 2
        for r in range(Rr):
            lo = u32[:, r * Do:r * Do + hw]
            hi = u32[:, r * Do + hw:(r + 1) * Do]
            z_ref[:, r * hw:(r + 1) * hw] = (lo | (hi << 16)).astype(jnp.int32)

    return pl.pallas_call(
        body,
        grid=(N // BN,),
        in_specs=[
            pl.BlockSpec((BN, D), lambda i: (i, 0)),
            pl.BlockSpec((Bb, D, Do), lambda i: (0, 0, 0)),
            pl.BlockSpec((Rr, Bb), lambda i: (0, 0)),
            pl.BlockSpec((D, Do), lambda i: (0, 0)),
        ],
        out_specs=[
            pl.BlockSpec((BN, Do), lambda i: (i, 0)),
            pl.BlockSpec((BN, Rr * (Do // 2)), lambda i: (i, 0)),
        ],
        out_shape=[
            jax.ShapeDtypeStruct((N, Do), jnp.float32),
            jax.ShapeDtypeStruct((N, Rr * (Do // 2)), jnp.int32),
        ],
        scratch_shapes=[pltpu.VMEM((D, Wtot), jnp.float32)],
    )(x, bases, coeff, W_self)


def _tc_pack(src, dst, rel, Rr):
    E_pad = src.shape[0]
    NR = E_pad // CH
    BR = 288
    s2 = src.reshape(NR, CH)
    d2 = dst.reshape(NR, CH)
    r2 = rel.reshape(NR, CH)

    def body(s_ref, d_ref, r_ref, o_ref):
        s = s_ref[...]
        d = d_ref[...]
        r = r_ref[...]
        o_ref[:, 0, :] = s * Rr + r
        o_ref[:, 1, :] = d * Rr + r
        o_ref[:, 2, :] = d

    return pl.pallas_call(
        body,
        grid=(NR // BR,),
        in_specs=[
            pl.BlockSpec((BR, CH), lambda i: (i, 0)),
            pl.BlockSpec((BR, CH), lambda i: (i, 0)),
            pl.BlockSpec((BR, CH), lambda i: (i, 0)),
        ],
        out_specs=pl.BlockSpec((BR, 3, CH), lambda i: (i, 0, 0)),
        out_shape=jax.ShapeDtypeStruct((NR, 3, CH), jnp.int32),
    )(s2, d2, r2)


def _sc_degree(idx3, ND):
    NR = idx3.shape[0]
    NCHUNK = NR // (NC * NS)
    SL = ND // NS
    mesh = plsc.VectorSubcoreMesh(core_axis_name="c", subcore_axis_name="s")

    @functools.partial(
        pl.kernel,
        mesh=mesh,
        out_type=jax.ShapeDtypeStruct((NC * ND,), jnp.float32),
        scratch_types=[
            pltpu.VMEM((3, 3, CH), jnp.int32),
            pltpu.VMEM((CH,), jnp.float32),
            pltpu.VMEM((SL,), jnp.float32),
            pltpu.VMEM_SHARED((ND,), jnp.float32),
            pltpu.SemaphoreType.DMA,
            pltpu.SemaphoreType.DMA,
            pltpu.SemaphoreType.DMA,
        ],
    )
    def k(idx_hbm, out_hbm, ix_v, ones_v, zb_v, deg_s, s0, s1, s2):
        cid = lax.axis_index("c")
        sid = lax.axis_index("s")
        wid = cid * NS + sid

        def zb_body(i, _):
            zb_v[pl.ds(i * LN, LN)] = jnp.zeros((LN,), jnp.float32)
            return 0

        lax.fori_loop(0, SL // LN, zb_body, 0)
        pltpu.sync_copy(zb_v, deg_s.at[pl.ds(sid * SL, SL)])

        def ones_body(i, _):
            ones_v[pl.ds(i * LN, LN)] = jnp.ones((LN,), jnp.float32)
            return 0

        lax.fori_loop(0, CH // LN, ones_body, 0)
        plsc.subcore_barrier()

        base = wid * NCHUNK
        sems = [s0, s1, s2]

        def fire(buf, sem, row):
            pltpu.sync_copy(idx_hbm.at[row], buf)
            pltpu.async_copy(ones_v, deg_s.at[buf.at[1]], sem, add=True)

        def drain(buf, sem):
            pltpu.make_async_copy(ones_v, deg_s.at[buf.at[1]], sem).wait()

        def triple(i, _):
            for u in range(3):
                c = 3 * i + u

                @pl.when(c >= 3)
                def _():
                    drain(ix_v.at[u], sems[u])

                fire(ix_v.at[u], sems[u], base + c)
            return 0

        lax.fori_loop(0, NCHUNK // 3, triple, 0)
        for u in range(3):
            drain(ix_v.at[u], sems[u])
        plsc.subcore_barrier()
        pltpu.sync_copy(deg_s.at[pl.ds(sid * SL, SL)],
                        out_hbm.at[pl.ds(cid * ND + sid * SL, SL)])

    return k(idx3)


def _tc_scale(degp, ND):
    d3 = degp.reshape(NC, ND // 128, 128)

    def body(d_ref, o_ref):
        s = d_ref[0] + d_ref[1]
        o_ref[...] = 1.0 / jnp.maximum(s, 1.0)

    out = pl.pallas_call(
        body,
        out_shape=jax.ShapeDtypeStruct((ND // 128, 128), jnp.float32),
    )(d3)
    return out.reshape(ND)


def _sc_edge(Zr, idx3, scale, NP):
    NR = idx3.shape[0]
    NCHUNK = NR // (NC * NS)
    DW = Zr.shape[1]
    D = DW * 2
    RPT = NP // NS
    mesh = plsc.VectorSubcoreMesh(core_axis_name="c", subcore_axis_name="s")

    @functools.partial(
        pl.kernel,
        mesh=mesh,
        out_type=jax.ShapeDtypeStruct((NC, NP, D), jnp.float32),
        compiler_params=pltpu.CompilerParams(use_tc_tiling_on_sc=False),
        scratch_types=[
            pltpu.VMEM((3, 3, CH), jnp.int32),
            pltpu.VMEM((3, CH), jnp.float32),
            pltpu.VMEM((3, CH, DW), jnp.int32),
            pltpu.VMEM((3, CH, D), jnp.float32),
            pltpu.VMEM_SHARED((NP, D), jnp.float32),
            pltpu.SemaphoreType.DMA,
            pltpu.SemaphoreType.DMA,
            pltpu.SemaphoreType.DMA,
            pltpu.SemaphoreType.DMA,
            pltpu.SemaphoreType.DMA,
            pltpu.SemaphoreType.DMA,
            pltpu.SemaphoreType.DMA,
            pltpu.SemaphoreType.DMA,
            pltpu.SemaphoreType.DMA,
        ],
    )
    def k(zr_hbm, idx_hbm, scale_hbm, out_hbm,
          ix_v, sc_v, gw_v, rw_v, acc_s, g0, g1, g2, t0, t1, t2, i0, i1, i2):
        cid = lax.axis_index("c")
        sid = lax.axis_index("s")
        wid = cid * NS + sid

        def z1(i, _):
            for jj in range(D // LN):
                rw_v[0, i, pl.ds(jj * LN, LN)] = jnp.zeros((LN,), jnp.float32)
            return 0

        lax.fori_loop(0, CH, z1, 0)
        nfull = RPT // CH
        tail = RPT - nfull * CH
        for kk in range(nfull):
            pltpu.sync_copy(rw_v.at[0],
                            acc_s.at[pl.ds(sid * RPT + kk * CH, CH)])
        if tail:
            pltpu.sync_copy(rw_v.at[0].at[pl.ds(0, tail)],
                            acc_s.at[pl.ds(sid * RPT + nfull * CH, tail)])
        plsc.subcore_barrier()

        base = wid * NCHUNK
        gsem = [g0, g1, g2]
        tsem = [t0, t1, t2]
        isem = [i0, i1, i2]

        def fire_idx(row, u):
            pltpu.async_copy(idx_hbm.at[row], ix_v.at[u], isem[u])

        def wait_idx(row, u):
            pltpu.make_async_copy(idx_hbm.at[row], ix_v.at[u], isem[u]).wait()

        def fire_gather(u):
            pltpu.async_copy(zr_hbm.at[ix_v.at[u, 0]], gw_v.at[u], gsem[u])
            pltpu.async_copy(scale_hbm.at[ix_v.at[u, 1]], sc_v.at[u], gsem[u])

        def drain_scatter(u):
            pltpu.make_async_copy(rw_v.at[u], acc_s.at[ix_v.at[u, 2]],
                                  tsem[u]).wait()

        def compute(u):
            pltpu.make_async_copy(zr_hbm.at[ix_v.at[u, 0]], gw_v.at[u],
                                  gsem[u]).wait()
            pltpu.make_async_copy(scale_hbm.at[ix_v.at[u, 1]], sc_v.at[u],
                                  gsem[u]).wait()
            mask_hi = jnp.full((LN,), -65536, jnp.int32)

            def scale_q(q, _):
                sv = sc_v[u, pl.ds(q * LN, LN)]
                for l in range(LN):
                    i = q * LN + l
                    s = sv[l]
                    for jj in range(DW // LN):
                        w = gw_v[u, i, pl.ds(jj * LN, LN)]
                        lo = lax.bitcast_convert_type(w << 16, jnp.float32)
                        hi = lax.bitcast_convert_type(w & mask_hi, jnp.float32)
                        rw_v[u, i, pl.ds(jj * LN, LN)] = lo * s
                        rw_v[u, i, pl.ds(DW + jj * LN, LN)] = hi * s
                return 0

            lax.fori_loop(0, CH // LN, scale_q, 0)
            pltpu.async_copy(rw_v.at[u], acc_s.at[ix_v.at[u, 2]], tsem[u],
                             add=True)

        pltpu.sync_copy(idx_hbm.at[base], ix_v.at[0])
        pltpu.sync_copy(idx_hbm.at[base + 1], ix_v.at[1])
        fire_gather(0)

        def triple(i, _):
            for u in range(3):
                c = 3 * i + u
                un1 = (u + 1) % 3
                un2 = (u + 2) % 3

                @pl.when(c >= 1)
                def _():
                    drain_scatter(un2)

                @pl.when(c + 2 < NCHUNK)
                def _():
                    fire_idx(base + c + 2, un2)

                @pl.when(c + 1 < NCHUNK)
                def _():
                    @pl.when(c >= 1)
                    def _():
                        wait_idx(base + c + 1, un1)

                    fire_gather(un1)

                compute(u)
            return 0

        lax.fori_loop(0, NCHUNK // 3, triple, 0)
        drain_scatter((NCHUNK - 1) % 3)
        plsc.subcore_barrier()
        pltpu.sync_copy(acc_s.at[pl.ds(sid * RPT, RPT)],
                        out_hbm.at[cid, pl.ds(sid * RPT, RPT)])

    return k(Zr, idx3, scale)


def _tc_final(h0, partial, NP):
    N, D = h0.shape
    BN = 1000

    def body(h_ref, p_ref, o_ref):
        o_ref[...] = jnp.maximum(h_ref[...] + p_ref[0] + p_ref[1], 0.0)

    return pl.pallas_call(
        body,
        grid=(N // BN,),
        in_specs=[
            pl.BlockSpec((BN, D), lambda i: (i, 0)),
            pl.BlockSpec((NC, BN, D), lambda i: (0, i, 0)),
        ],
        out_specs=pl.BlockSpec((BN, D), lambda i: (i, 0)),
        out_shape=jax.ShapeDtypeStruct((N, D), jnp.float32),
    )(h0, partial)


def kernel(x, edge_index, relation_index, bases, coefficients, W_self):
    N, D = x.shape
    Rr = coefficients.shape[0]
    Do = W_self.shape[1]
    E = relation_index.shape[0]

    hidden0, Zw2 = _tc_transform(x, bases, coefficients, W_self)
    Zw = Zw2.reshape(N * Rr, Do // 2)

    NW = NC * NS
    EQ = NW * CH * 3
    E_pad = ((E + EQ - 1) // EQ) * EQ
    pad = E_pad - E
    src = jnp.concatenate([edge_index[0], jnp.zeros((pad,), jnp.int32)])
    dst = jnp.concatenate([edge_index[1], jnp.full((pad,), N, jnp.int32)])
    rel = jnp.concatenate([relation_index, jnp.zeros((pad,), jnp.int32)])
    idx3 = _tc_pack(src, dst, rel, Rr)

    ND = ((N * Rr + 1 + NS * 128 - 1) // (NS * 128)) * (NS * 128)
    degp = _sc_degree(idx3, ND)
    scale = _tc_scale(degp, ND)

    NP = ((N + 1 + NS * 8 - 1) // (NS * 8)) * (NS * 8)
    partial = _sc_edge(Zw, idx3, scale, NP)
    return _tc_final(hidden0, partial, NP)


# speedup vs baseline: 1.1786x; 1.1786x over previous
import functools

import jax
import jax.numpy as jnp
from jax import lax
from jax.experimental import pallas as pl
from jax.experimental.pallas import tpu as pltpu
from jax.experimental.pallas import tpu_sc as plsc

NC = 2
NS = 16
LN = 16
CH = 112


def _tc_transform(x, bases, coeff, W_self):
    N, D = x.shape
    Bb = bases.shape[0]
    Rr = coeff.shape[0]
    Do = W_self.shape[1]
    BN = 1000
    Wtot = Do * (Rr + 1)

    def body(x_ref, b_ref, c_ref, w_ref, h_ref, z_ref, wcat_ref):
        @pl.when(pl.program_id(0) == 0)
        def _():
            wcat_ref[:, 0:Do] = w_ref[...]
            c = c_ref[...]
            for r in range(Rr):
                acc = c[r, 0] * b_ref[0]
                for b in range(1, Bb):
                    acc = acc + c[r, b] * b_ref[b]
                wcat_ref[:, (r + 1) * Do:(r + 2) * Do] = acc

        y = jnp.dot(x_ref[...], wcat_ref[...], preferred_element_type=jnp.float32)
        h_ref[...] = y[:, 0:Do]
        z_ref[...] = y[:, Do:]

    return pl.pallas_call(
        body,
        grid=(N // BN,),
        in_specs=[
            pl.BlockSpec((BN, D), lambda i: (i, 0)),
            pl.BlockSpec((Bb, D, Do), lambda i: (0, 0, 0)),
            pl.BlockSpec((Rr, Bb), lambda i: (0, 0)),
            pl.BlockSpec((D, Do), lambda i: (0, 0)),
        ],
        out_specs=[
            pl.BlockSpec((BN, Do), lambda i: (i, 0)),
            pl.BlockSpec((BN, Rr * Do), lambda i: (i, 0)),
        ],
        out_shape=[
            jax.ShapeDtypeStruct((N, Do), jnp.float32),
            jax.ShapeDtypeStruct((N, Rr * Do), jnp.float32),
        ],
        scratch_shapes=[pltpu.VMEM((D, Wtot), jnp.float32)],
    )(x, bases, coeff, W_self)


def _tc_pack(src, dst, rel, Rr):
    E_pad = src.shape[0]
    NR = E_pad // CH
    BR = 288
    s2 = src.reshape(NR, CH)
    d2 = dst.reshape(NR, CH)
    r2 = rel.reshape(NR, CH)

    def body(s_ref, d_ref, r_ref, o_ref):
        s = s_ref[...]
        d = d_ref[...]
        r = r_ref[...]
        o_ref[:, 0, :] = s * Rr + r
        o_ref[:, 1, :] = d * Rr + r
        o_ref[:, 2, :] = d

    return pl.pallas_call(
        body,
        grid=(NR // BR,),
        in_specs=[
            pl.BlockSpec((BR, CH), lambda i: (i, 0)),
            pl.BlockSpec((BR, CH), lambda i: (i, 0)),
            pl.BlockSpec((BR, CH), lambda i: (i, 0)),
        ],
        out_specs=pl.BlockSpec((BR, 3, CH), lambda i: (i, 0, 0)),
        out_shape=jax.ShapeDtypeStruct((NR, 3, CH), jnp.int32),
    )(s2, d2, r2)


def _sc_degree(idx3, ND):
    NR = idx3.shape[0]
    NCHUNK = NR // (NC * NS)
    SL = ND // NS
    mesh = plsc.VectorSubcoreMesh(core_axis_name="c", subcore_axis_name="s")

    @functools.partial(
        pl.kernel,
        mesh=mesh,
        out_type=jax.ShapeDtypeStruct((NC * ND,), jnp.float32),
        scratch_types=[
            pltpu.VMEM((3, 3, CH), jnp.int32),
            pltpu.VMEM((CH,), jnp.float32),
            pltpu.VMEM((SL,), jnp.float32),
            pltpu.VMEM_SHARED((ND,), jnp.float32),
            pltpu.SemaphoreType.DMA,
            pltpu.SemaphoreType.DMA,
            pltpu.SemaphoreType.DMA,
        ],
    )
    def k(idx_hbm, out_hbm, ix_v, ones_v, zb_v, deg_s, s0, s1, s2):
        cid = lax.axis_index("c")
        sid = lax.axis_index("s")
        wid = cid * NS + sid

        def zb_body(i, _):
            zb_v[pl.ds(i * LN, LN)] = jnp.zeros((LN,), jnp.float32)
            return 0

        lax.fori_loop(0, SL // LN, zb_body, 0)
        pltpu.sync_copy(zb_v, deg_s.at[pl.ds(sid * SL, SL)])

        def ones_body(i, _):
            ones_v[pl.ds(i * LN, LN)] = jnp.ones((LN,), jnp.float32)
            return 0

        lax.fori_loop(0, CH // LN, ones_body, 0)
        plsc.subcore_barrier()

        base = wid * NCHUNK
        sems = [s0, s1, s2]

        def fire(buf, sem, row):
            pltpu.sync_copy(idx_hbm.at[row], buf)
            pltpu.async_copy(ones_v, deg_s.at[buf.at[1]], sem, add=True)

        def drain(buf, sem):
            pltpu.make_async_copy(ones_v, deg_s.at[buf.at[1]], sem).wait()

        def triple(i, _):
            for u in range(3):
                c = 3 * i + u

                @pl.when(c >= 3)
                def _():
                    drain(ix_v.at[u], sems[u])

                fire(ix_v.at[u], sems[u], base + c)
            return 0

        lax.fori_loop(0, NCHUNK // 3, triple, 0)
        for u in range(3):
            drain(ix_v.at[u], sems[u])
        plsc.subcore_barrier()
        pltpu.sync_copy(deg_s.at[pl.ds(sid * SL, SL)],
                        out_hbm.at[pl.ds(cid * ND + sid * SL, SL)])

    return k(idx3)


def _tc_scale(degp, ND):
    d3 = degp.reshape(NC, ND // 128, 128)

    def body(d_ref, o_ref):
        s = d_ref[0] + d_ref[1]
        o_ref[...] = 1.0 / jnp.maximum(s, 1.0)

    out = pl.pallas_call(
        body,
        out_shape=jax.ShapeDtypeStruct((ND // 128, 128), jnp.float32),
    )(d3)
    return out.reshape(ND)


def _sc_edge(Zr, idx3, scale, NP):
    NR = idx3.shape[0]
    NCHUNK = NR // (NC * NS)
    D = Zr.shape[1]
    RPT = NP // NS
    mesh = plsc.VectorSubcoreMesh(core_axis_name="c", subcore_axis_name="s")

    @functools.partial(
        pl.kernel,
        mesh=mesh,
        out_type=jax.ShapeDtypeStruct((NC, NP, D), jnp.float32),
        scratch_types=[
            pltpu.VMEM((3, 3, CH), jnp.int32),
            pltpu.VMEM((3, CH), jnp.float32),
            pltpu.VMEM((3, CH, D), jnp.float32),
            pltpu.VMEM_SHARED((NP, D), jnp.float32),
            pltpu.SemaphoreType.DMA,
            pltpu.SemaphoreType.DMA,
            pltpu.SemaphoreType.DMA,
            pltpu.SemaphoreType.DMA,
            pltpu.SemaphoreType.DMA,
            pltpu.SemaphoreType.DMA,
            pltpu.SemaphoreType.DMA,
            pltpu.SemaphoreType.DMA,
            pltpu.SemaphoreType.DMA,
        ],
    )
    def k(zr_hbm, idx_hbm, scale_hbm, out_hbm,
          ix_v, sc_v, rw_v, acc_s, g0, g1, g2, t0, t1, t2, i0, i1, i2):
        cid = lax.axis_index("c")
        sid = lax.axis_index("s")
        wid = cid * NS + sid

        def z1(i, _):
            for jj in range(D // LN):
                rw_v[0, i, pl.ds(jj * LN, LN)] = jnp.zeros((LN,), jnp.float32)
            return 0

        lax.fori_loop(0, CH, z1, 0)
        nfull = RPT // CH
        tail = RPT - nfull * CH
        for kk in range(nfull):
            pltpu.sync_copy(rw_v.at[0],
                            acc_s.at[pl.ds(sid * RPT + kk * CH, CH)])
        if tail:
            pltpu.sync_copy(rw_v.at[0].at[pl.ds(0, tail)],
                            acc_s.at[pl.ds(sid * RPT + nfull * CH, tail)])
        plsc.subcore_barrier()

        base = wid * NCHUNK
        gsem = [g0, g1, g2]
        tsem = [t0, t1, t2]
        isem = [i0, i1, i2]

        def fire_idx(row, u):
            pltpu.async_copy(idx_hbm.at[row], ix_v.at[u], isem[u])

        def wait_idx(row, u):
            pltpu.make_async_copy(idx_hbm.at[row], ix_v.at[u], isem[u]).wait()

        def fire_gather(u):
            pltpu.async_copy(zr_hbm.at[ix_v.at[u, 0]], rw_v.at[u], gsem[u])
            pltpu.async_copy(scale_hbm.at[ix_v.at[u, 1]], sc_v.at[u], gsem[u])

        def drain_scatter(u):
            pltpu.make_async_copy(rw_v.at[u], acc_s.at[ix_v.at[u, 2]],
                                  tsem[u]).wait()

        def compute(u):
            pltpu.make_async_copy(zr_hbm.at[ix_v.at[u, 0]], rw_v.at[u],
                                  gsem[u]).wait()
            pltpu.make_async_copy(scale_hbm.at[ix_v.at[u, 1]], sc_v.at[u],
                                  gsem[u]).wait()

            def scale_q(q, _):
                sv = sc_v[u, pl.ds(q * LN, LN)]
                for l in range(LN):
                    i = q * LN + l
                    s = sv[l]
                    for jj in range(D // LN):
                        sl = pl.ds(jj * LN, LN)
                        rw_v[u, i, sl] = rw_v[u, i, sl] * s
                return 0

            lax.fori_loop(0, CH // LN, scale_q, 0)
            pltpu.async_copy(rw_v.at[u], acc_s.at[ix_v.at[u, 2]], tsem[u],
                             add=True)

        pltpu.sync_copy(idx_hbm.at[base], ix_v.at[0])
        pltpu.sync_copy(idx_hbm.at[base + 1], ix_v.at[1])
        fire_gather(0)

        def triple(i, _):
            for u in range(3):
                c = 3 * i + u
                un1 = (u + 1) % 3
                un2 = (u + 2) % 3

                @pl.when(c >= 1)
                def _():
                    drain_scatter(un2)

                @pl.when(c + 2 < NCHUNK)
                def _():
                    fire_idx(base + c + 2, un2)

                @pl.when(c + 1 < NCHUNK)
                def _():
                    @pl.when(c >= 1)
                    def _():
                        wait_idx(base + c + 1, un1)

                    fire_gather(un1)

                compute(u)
            return 0

        lax.fori_loop(0, NCHUNK // 3, triple, 0)
        drain_scatter((NCHUNK - 1) % 3)
        plsc.subcore_barrier()
        pltpu.sync_copy(acc_s.at[pl.ds(sid * RPT, RPT)],
                        out_hbm.at[cid, pl.ds(sid * RPT, RPT)])

    return k(Zr, idx3, scale)


def _tc_final(h0, partial, NP):
    N, D = h0.shape
    BN = 1000

    def body(h_ref, p_ref, o_ref):
        o_ref[...] = jnp.maximum(h_ref[...] + p_ref[0] + p_ref[1], 0.0)

    return pl.pallas_call(
        body,
        grid=(N // BN,),
        in_specs=[
            pl.BlockSpec((BN, D), lambda i: (i, 0)),
            pl.BlockSpec((NC, BN, D), lambda i: (0, i, 0)),
        ],
        out_specs=pl.BlockSpec((BN, D), lambda i: (i, 0)),
        out_shape=jax.ShapeDtypeStruct((N, D), jnp.float32),
    )(h0, partial)


def kernel(x, edge_index, relation_index, bases, coefficients, W_self):
    N, D = x.shape
    Rr = coefficients.shape[0]
    Do = W_self.shape[1]
    E = relation_index.shape[0]

    hidden0, Zr2 = _tc_transform(x, bases, coefficients, W_self)
    Zr = Zr2.reshape(N * Rr, Do)

    NW = NC * NS
    EQ = NW * CH * 3
    E_pad = ((E + EQ - 1) // EQ) * EQ
    pad = E_pad - E
    src = jnp.concatenate([edge_index[0], jnp.zeros((pad,), jnp.int32)])
    dst = jnp.concatenate([edge_index[1], jnp.full((pad,), N, jnp.int32)])
    rel = jnp.concatenate([relation_index, jnp.zeros((pad,), jnp.int32)])
    idx3 = _tc_pack(src, dst, rel, Rr)

    ND = ((N * Rr + 1 + NS * 128 - 1) // (NS * 128)) * (NS * 128)
    degp = _sc_degree(idx3, ND)
    scale = _tc_scale(degp, ND)

    NP = ((N + 1 + NS * 8 - 1) // (NS * 8)) * (NS * 8)
    partial = _sc_edge(Zr, idx3, scale, NP)
    return _tc_final(hidden0, partial, NP)


# speedup vs baseline: 1.1950x; 1.0139x over previous
import functools

import jax
import jax.numpy as jnp
from jax import lax
from jax.experimental import pallas as pl
from jax.experimental.pallas import tpu as pltpu
from jax.experimental.pallas import tpu_sc as plsc

NC = 2
NS = 16
LN = 16
CH = 112


def _tc_transform(x, bases, coeff, W_self):
    N, D = x.shape
    Bb = bases.shape[0]
    Rr = coeff.shape[0]
    Do = W_self.shape[1]
    BN = 1000
    Wtot = Do * (Rr + 1)

    def body(x_ref, b_ref, c_ref, w_ref, h_ref, z_ref, wcat_ref):
        @pl.when(pl.program_id(0) == 0)
        def _():
            wcat_ref[:, 0:Do] = w_ref[...]
            c = c_ref[...]
            for r in range(Rr):
                acc = c[r, 0] * b_ref[0]
                for b in range(1, Bb):
                    acc = acc + c[r, b] * b_ref[b]
                wcat_ref[:, (r + 1) * Do:(r + 2) * Do] = acc

        y = jnp.dot(x_ref[...], wcat_ref[...], preferred_element_type=jnp.float32)
        h_ref[...] = y[:, 0:Do]
        z_ref[...] = y[:, Do:]

    return pl.pallas_call(
        body,
        grid=(N // BN,),
        in_specs=[
            pl.BlockSpec((BN, D), lambda i: (i, 0)),
            pl.BlockSpec((Bb, D, Do), lambda i: (0, 0, 0)),
            pl.BlockSpec((Rr, Bb), lambda i: (0, 0)),
            pl.BlockSpec((D, Do), lambda i: (0, 0)),
        ],
        out_specs=[
            pl.BlockSpec((BN, Do), lambda i: (i, 0)),
            pl.BlockSpec((BN, Rr * Do), lambda i: (i, 0)),
        ],
        out_shape=[
            jax.ShapeDtypeStruct((N, Do), jnp.float32),
            jax.ShapeDtypeStruct((N, Rr * Do), jnp.float32),
        ],
        scratch_shapes=[pltpu.VMEM((D, Wtot), jnp.float32)],
    )(x, bases, coeff, W_self)


def _tc_pack(src, dst, rel, Rr):
    E_pad = src.shape[0]
    NR = E_pad // CH
    BR = 288
    s2 = src.reshape(NR, CH)
    d2 = dst.reshape(NR, CH)
    r2 = rel.reshape(NR, CH)

    def body(s_ref, d_ref, r_ref, o_ref):
        s = s_ref[...]
        d = d_ref[...]
        r = r_ref[...]
        o_ref[:, 0, :] = s * Rr + r
        o_ref[:, 1, :] = d * Rr + r
        o_ref[:, 2, :] = d

    return pl.pallas_call(
        body,
        grid=(NR // BR,),
        in_specs=[
            pl.BlockSpec((BR, CH), lambda i: (i, 0)),
            pl.BlockSpec((BR, CH), lambda i: (i, 0)),
            pl.BlockSpec((BR, CH), lambda i: (i, 0)),
        ],
        out_specs=pl.BlockSpec((BR, 3, CH), lambda i: (i, 0, 0)),
        out_shape=jax.ShapeDtypeStruct((NR, 3, CH), jnp.int32),
    )(s2, d2, r2)


def _sc_degree(idx3, ND):
    NR = idx3.shape[0]
    NCHUNK = NR // (NC * NS)
    SL = ND // NS
    mesh = plsc.VectorSubcoreMesh(core_axis_name="c", subcore_axis_name="s")

    @functools.partial(
        pl.kernel,
        mesh=mesh,
        out_type=jax.ShapeDtypeStruct((NC * ND,), jnp.float32),
        scratch_types=[
            pltpu.VMEM((6, 3, CH), jnp.int32),
            pltpu.VMEM((CH,), jnp.float32),
            pltpu.VMEM((SL,), jnp.float32),
            pltpu.VMEM_SHARED((ND,), jnp.float32),
            pltpu.SemaphoreType.DMA,
            pltpu.SemaphoreType.DMA,
            pltpu.SemaphoreType.DMA,
            pltpu.SemaphoreType.DMA,
            pltpu.SemaphoreType.DMA,
            pltpu.SemaphoreType.DMA,
        ],
    )
    def k(idx_hbm, out_hbm, ix_v, ones_v, zb_v, deg_s,
          s0, s1, s2, j0, j1, j2):
        cid = lax.axis_index("c")
        sid = lax.axis_index("s")
        wid = cid * NS + sid

        def zb_body(i, _):
            zb_v[pl.ds(i * LN, LN)] = jnp.zeros((LN,), jnp.float32)
            return 0

        lax.fori_loop(0, SL // LN, zb_body, 0)
        pltpu.sync_copy(zb_v, deg_s.at[pl.ds(sid * SL, SL)])

        def ones_body(i, _):
            ones_v[pl.ds(i * LN, LN)] = jnp.ones((LN,), jnp.float32)
            return 0

        lax.fori_loop(0, CH // LN, ones_body, 0)
        plsc.subcore_barrier()

        base = wid * NCHUNK
        sems = [s0, s1, s2]
        isem = [j0, j1, j2]

        def hexa(i, _):
            for u6 in range(6):
                c = 6 * i + u6
                u3 = u6 % 3
                buf = ix_v.at[u6]

                @pl.when(c >= 3)
                def _():
                    pltpu.make_async_copy(
                        ones_v, deg_s.at[ix_v.at[(u6 + 3) % 6, 1]],
                        sems[u3]).wait()
                    pltpu.make_async_copy(idx_hbm.at[base + c], buf,
                                          isem[u3]).wait()

                @pl.when(c < 3)
                def _():
                    pltpu.sync_copy(idx_hbm.at[base + c], buf)

                pltpu.async_copy(ones_v, deg_s.at[buf.at[1]], sems[u3],
                                 add=True)

                @pl.when(c + 3 < NCHUNK)
                def _():
                    pltpu.async_copy(idx_hbm.at[base + c + 3],
                                     ix_v.at[(u6 + 3) % 6], isem[u3])
            return 0

        lax.fori_loop(0, NCHUNK // 6, hexa, 0)
        for u6 in range(3):
            pltpu.make_async_copy(ones_v, deg_s.at[ix_v.at[3 + u6, 1]],
                                  sems[u6]).wait()
        plsc.subcore_barrier()
        pltpu.sync_copy(deg_s.at[pl.ds(sid * SL, SL)],
                        out_hbm.at[pl.ds(cid * ND + sid * SL, SL)])

    return k(idx3)


def _tc_scale(degp, ND):
    d3 = degp.reshape(NC, ND // 128, 128)

    def body(d_ref, o_ref):
        s = d_ref[0] + d_ref[1]
        o_ref[...] = 1.0 / jnp.maximum(s, 1.0)

    out = pl.pallas_call(
        body,
        out_shape=jax.ShapeDtypeStruct((ND // 128, 128), jnp.float32),
    )(d3)
    return out.reshape(ND)


def _sc_edge(Zr, idx3, scale, NP):
    NR = idx3.shape[0]
    NCHUNK = NR // (NC * NS)
    D = Zr.shape[1]
    RPT = NP // NS
    mesh = plsc.VectorSubcoreMesh(core_axis_name="c", subcore_axis_name="s")

    @functools.partial(
        pl.kernel,
        mesh=mesh,
        out_type=jax.ShapeDtypeStruct((NC, NP, D), jnp.float32),
        scratch_types=[
            pltpu.VMEM((3, 3, CH), jnp.int32),
            pltpu.VMEM((3, CH), jnp.float32),
            pltpu.VMEM((3, CH, D), jnp.float32),
            pltpu.VMEM_SHARED((NP, D), jnp.float32),
            pltpu.SemaphoreType.DMA,
            pltpu.SemaphoreType.DMA,
            pltpu.SemaphoreType.DMA,
            pltpu.SemaphoreType.DMA,
            pltpu.SemaphoreType.DMA,
            pltpu.SemaphoreType.DMA,
            pltpu.SemaphoreType.DMA,
            pltpu.SemaphoreType.DMA,
            pltpu.SemaphoreType.DMA,
        ],
    )
    def k(zr_hbm, idx_hbm, scale_hbm, out_hbm,
          ix_v, sc_v, rw_v, acc_s, g0, g1, g2, t0, t1, t2, i0, i1, i2):
        cid = lax.axis_index("c")
        sid = lax.axis_index("s")
        wid = cid * NS + sid

        def z1(i, _):
            for jj in range(D // LN):
                rw_v[0, i, pl.ds(jj * LN, LN)] = jnp.zeros((LN,), jnp.float32)
            return 0

        lax.fori_loop(0, CH, z1, 0)
        nfull = RPT // CH
        tail = RPT - nfull * CH
        for kk in range(nfull):
            pltpu.sync_copy(rw_v.at[0],
                            acc_s.at[pl.ds(sid * RPT + kk * CH, CH)])
        if tail:
            pltpu.sync_copy(rw_v.at[0].at[pl.ds(0, tail)],
                            acc_s.at[pl.ds(sid * RPT + nfull * CH, tail)])
        plsc.subcore_barrier()

        base = wid * NCHUNK
        gsem = [g0, g1, g2]
        tsem = [t0, t1, t2]
        isem = [i0, i1, i2]

        def fire_idx(row, u):
            pltpu.async_copy(idx_hbm.at[row], ix_v.at[u], isem[u])

        def wait_idx(row, u):
            pltpu.make_async_copy(idx_hbm.at[row], ix_v.at[u], isem[u]).wait()

        def fire_gather(u):
            pltpu.async_copy(zr_hbm.at[ix_v.at[u, 0]], rw_v.at[u], gsem[u])
            pltpu.async_copy(scale_hbm.at[ix_v.at[u, 1]], sc_v.at[u], gsem[u])

        def drain_scatter(u):
            pltpu.make_async_copy(rw_v.at[u], acc_s.at[ix_v.at[u, 2]],
                                  tsem[u]).wait()

        def compute(u):
            pltpu.make_async_copy(zr_hbm.at[ix_v.at[u, 0]], rw_v.at[u],
                                  gsem[u]).wait()
            pltpu.make_async_copy(scale_hbm.at[ix_v.at[u, 1]], sc_v.at[u],
                                  gsem[u]).wait()

            def scale_q(q, _):
                sv = sc_v[u, pl.ds(q * LN, LN)]
                for l in range(LN):
                    i = q * LN + l
                    s = sv[l]
                    for jj in range(D // LN):
                        sl = pl.ds(jj * LN, LN)
                        rw_v[u, i, sl] = rw_v[u, i, sl] * s
                return 0

            lax.fori_loop(0, CH // LN, scale_q, 0)
            pltpu.async_copy(rw_v.at[u], acc_s.at[ix_v.at[u, 2]], tsem[u],
                             add=True)

        pltpu.sync_copy(idx_hbm.at[base], ix_v.at[0])
        pltpu.sync_copy(idx_hbm.at[base + 1], ix_v.at[1])
        fire_gather(0)

        def triple(i, _):
            for u in range(3):
                c = 3 * i + u
                un1 = (u + 1) % 3
                un2 = (u + 2) % 3

                @pl.when(c >= 1)
                def _():
                    drain_scatter(un2)

                @pl.when(c + 2 < NCHUNK)
                def _():
                    fire_idx(base + c + 2, un2)

                @pl.when(c + 1 < NCHUNK)
                def _():
                    @pl.when(c >= 1)
                    def _():
                        wait_idx(base + c + 1, un1)

                    fire_gather(un1)

                compute(u)
            return 0

        lax.fori_loop(0, NCHUNK // 3, triple, 0)
        drain_scatter((NCHUNK - 1) % 3)
        plsc.subcore_barrier()
        pltpu.sync_copy(acc_s.at[pl.ds(sid * RPT, RPT)],
                        out_hbm.at[cid, pl.ds(sid * RPT, RPT)])

    return k(Zr, idx3, scale)


def _tc_final(h0, partial, NP):
    N, D = h0.shape
    BN = 1000

    def body(h_ref, p_ref, o_ref):
        o_ref[...] = jnp.maximum(h_ref[...] + p_ref[0] + p_ref[1], 0.0)

    return pl.pallas_call(
        body,
        grid=(N // BN,),
        in_specs=[
            pl.BlockSpec((BN, D), lambda i: (i, 0)),
            pl.BlockSpec((NC, BN, D), lambda i: (0, i, 0)),
        ],
        out_specs=pl.BlockSpec((BN, D), lambda i: (i, 0)),
        out_shape=jax.ShapeDtypeStruct((N, D), jnp.float32),
    )(h0, partial)


def kernel(x, edge_index, relation_index, bases, coefficients, W_self):
    N, D = x.shape
    Rr = coefficients.shape[0]
    Do = W_self.shape[1]
    E = relation_index.shape[0]

    hidden0, Zr2 = _tc_transform(x, bases, coefficients, W_self)
    Zr = Zr2.reshape(N * Rr, Do)

    NW = NC * NS
    EQ = NW * CH * 3
    E_pad = ((E + EQ - 1) // EQ) * EQ
    pad = E_pad - E
    src = jnp.concatenate([edge_index[0], jnp.zeros((pad,), jnp.int32)])
    dst = jnp.concatenate([edge_index[1], jnp.full((pad,), N, jnp.int32)])
    rel = jnp.concatenate([relation_index, jnp.zeros((pad,), jnp.int32)])
    idx3 = _tc_pack(src, dst, rel, Rr)

    ND = ((N * Rr + 1 + NS * 128 - 1) // (NS * 128)) * (NS * 128)
    degp = _sc_degree(idx3, ND)
    scale = _tc_scale(degp, ND)

    NP = ((N + 1 + NS * 8 - 1) // (NS * 8)) * (NS * 8)
    partial = _sc_edge(Zr, idx3, scale, NP)
    return _tc_final(hidden0, partial, NP)
